# baseline (speedup 1.0000x reference)
import functools

import jax
import jax.numpy as jnp
from jax import lax
from jax.experimental import pallas as pl
from jax.experimental.pallas import tpu as pltpu
from jax.experimental.pallas import tpu_sc as plsc

_NS = [10000, 5000, 2500, 1250, 625]
_ES = [320000, 160000, 80000, 40000, 20000]
_NPAD = [10240, 5120, 2560, 1280, 640]

_NCORE, _NSUB = 2, 16
_NWORK = _NCORE * _NSUB
_K = 128


def _round_up(x, m):
    return (x + m - 1) // m * m


def _mesh():
    return plsc.VectorSubcoreMesh(
        core_axis_name="c", subcore_axis_name="s",
        num_cores=_NCORE, num_subcores=_NSUB)


@functools.lru_cache(maxsize=None)
def _sc_edge_agg(ep, np_dst, ch):
    nrows = ep // _K
    rw = nrows // _NWORK
    rows_sub = np_dst // _NSUB
    _NB = 8 if ch <= 80 else 4
    _AH = _NB // 2

    def body(*refs):
        x_hbm, src_hbm, dst_hbm, zeros_hbm, out_hbm, sidx, didx = refs[:7]
        rows = list(refs[7:7 + _NB])
        gs = list(refs[7 + _NB:7 + 2 * _NB])
        ss = list(refs[7 + 2 * _NB:7 + 3 * _NB])
        acc = refs[7 + 3 * _NB]
        c = lax.axis_index("c")
        s = lax.axis_index("s")
        w = c * _NSUB + s
        pltpu.sync_copy(zeros_hbm.at[pl.ds(s * rows_sub, rows_sub)],
                        acc.at[pl.ds(s * rows_sub, rows_sub)])
        pltpu.sync_copy(src_hbm.at[pl.ds(w * rw, rw)], sidx)
        pltpu.sync_copy(dst_hbm.at[pl.ds(w * rw, rw)], didx)
        plsc.subcore_barrier()

        def g_start(j, b):
            pltpu.async_copy(x_hbm.at[sidx.at[j]], rows[b], gs[b])

        def g_wait(j, b):
            pltpu.make_async_copy(x_hbm.at[sidx.at[j]], rows[b],
                                  gs[b]).wait()

        def s_start(j, b):
            pltpu.async_copy(rows[b], acc.at[didx.at[j]], ss[b], add=True)

        def s_wait(j, b):
            pltpu.make_async_copy(rows[b], acc.at[didx.at[j]],
                                  ss[b]).wait()

        def step(j, b, static_j=None):
            g_wait(j, b)
            s_start(j, b)
            jm = j - _AH
            jp = j + _AH
            if static_j is None:
                @pl.when(jm >= 0)
                def _():
                    s_wait(jm, (b - _AH) % _NB)

                @pl.when(jp < rw)
                def _():
                    g_start(jp, (b + _AH) % _NB)
            else:
                if static_j - _AH >= 0:
                    s_wait(jm, (b - _AH) % _NB)
                if static_j + _AH < rw:
                    g_start(jp, (b + _AH) % _NB)

        for b0 in range(min(_AH, rw)):
            g_start(b0, b0)
        nq = rw // _NB

        def quad(q, carry):
            j0 = q * _NB
            for b in range(_NB):
                step(j0 + b, b)
            return carry
        lax.fori_loop(0, nq, quad, 0)
        for j in range(nq * _NB, rw):
            step(j, j % _NB, static_j=j)
        for j in range(max(0, rw - _AH), rw):
            s_wait(j, j % _NB)

        plsc.subcore_barrier()
        pltpu.sync_copy(acc.at[pl.ds(s * rows_sub, rows_sub)],
                        out_hbm.at[c, pl.ds(s * rows_sub, rows_sub)])

    scratch_types = (
        [pltpu.VMEM((rw, _K), jnp.int32)] * 2
        + [pltpu.VMEM((_K, ch), jnp.float32)] * _NB
        + [pltpu.SemaphoreType.DMA] * (2 * _NB)
        + [pltpu.VMEM_SHARED((np_dst, ch), jnp.float32)]
    )

    return pl.kernel(
        body,
        out_type=jax.ShapeDtypeStruct((2, np_dst, ch), jnp.float32),
        mesh=_mesh(),
        scratch_types=scratch_types,
        compiler_params=pltpu.CompilerParams(use_tc_tiling_on_sc=False),
        name=f"sc_edge_agg_e{ep}_n{np_dst}_c{ch}",
    )


@functools.lru_cache(maxsize=None)
def _sc_seg_sum(ep, np_dst, ch, gather):
    ew = ep // _NWORK
    nfull, tail = ew // _K, ew % _K
    rows_sub = np_dst // _NSUB

    def body(*refs):
        if gather:
            x_hbm, src_hbm, dst_hbm, zeros_hbm, out_hbm = refs[:5]
            scratch = refs[5:]
        else:
            x_hbm, dst_hbm, zeros_hbm, out_hbm = refs[:4]
            src_hbm = None
            scratch = refs[4:]
        sidx_v, didx_v, rows_v = scratch[:3]
        scratch = scratch[3:]
        if tail:
            sidx_t, didx_t, rows_t = scratch[:3]
            scratch = scratch[3:]
        acc = scratch[0]

        c = lax.axis_index("c")
        s = lax.axis_index("s")
        base_w = (c * _NSUB + s) * ew

        pltpu.sync_copy(zeros_hbm.at[pl.ds(s * rows_sub, rows_sub)],
                        acc.at[pl.ds(s * rows_sub, rows_sub)])
        plsc.subcore_barrier()

        if nfull:
            def step(j, carry):
                base = base_w + j * _K
                pltpu.sync_copy(dst_hbm.at[pl.ds(base, _K)], didx_v)
                if gather:
                    pltpu.sync_copy(src_hbm.at[pl.ds(base, _K)], sidx_v)
                    pltpu.sync_copy(x_hbm.at[sidx_v], rows_v)
                else:
                    pltpu.sync_copy(x_hbm.at[pl.ds(base, _K)], rows_v)
                pltpu.sync_copy(rows_v, acc.at[didx_v], add=True)
                return carry
            lax.fori_loop(0, nfull, step, 0)
        if tail:
            base = base_w + nfull * _K
            pltpu.sync_copy(dst_hbm.at[pl.ds(base, tail)], didx_t)
            if gather:
                pltpu.sync_copy(src_hbm.at[pl.ds(base, tail)], sidx_t)
                pltpu.sync_copy(x_hbm.at[sidx_t], rows_t)
            else:
                pltpu.sync_copy(x_hbm.at[pl.ds(base, tail)], rows_t)
            pltpu.sync_copy(rows_t, acc.at[didx_t], add=True)

        plsc.subcore_barrier()
        pltpu.sync_copy(acc.at[pl.ds(s * rows_sub, rows_sub)],
                        out_hbm.at[c, pl.ds(s * rows_sub, rows_sub)])

    scratch_types = [
        pltpu.VMEM((_K,), jnp.int32),
        pltpu.VMEM((_K,), jnp.int32),
        pltpu.VMEM((_K, ch), jnp.float32),
    ]
    if tail:
        scratch_types += [
            pltpu.VMEM((tail,), jnp.int32),
            pltpu.VMEM((tail,), jnp.int32),
            pltpu.VMEM((tail, ch), jnp.float32),
        ]
    scratch_types.append(pltpu.VMEM_SHARED((np_dst, ch), jnp.float32))

    return pl.kernel(
        body,
        out_type=jax.ShapeDtypeStruct((2, np_dst, ch), jnp.float32),
        mesh=_mesh(),
        scratch_types=scratch_types,
        compiler_params=pltpu.CompilerParams(use_tc_tiling_on_sc=False),
        name=f"sc_seg_sum_e{ep}_n{np_dst}_c{ch}_{int(gather)}",
    )


@functools.lru_cache(maxsize=None)
def _sc_row_gather(np_out, ch):
    ew = np_out // _NWORK
    nfull, tail = ew // _K, ew % _K

    def body(*refs):
        tab_hbm, idx_hbm, out_hbm = refs[:3]
        scratch = refs[3:]
        idx_v, rows_v = scratch[:2]
        if tail:
            idx_t, rows_t = scratch[2:4]
        c = lax.axis_index("c")
        s = lax.axis_index("s")
        base_w = (c * _NSUB + s) * ew
        if nfull:
            def step(j, carry):
                base = base_w + j * _K
                pltpu.sync_copy(idx_hbm.at[pl.ds(base, _K)], idx_v)
                pltpu.sync_copy(tab_hbm.at[idx_v], rows_v)
                pltpu.sync_copy(rows_v, out_hbm.at[pl.ds(base, _K)])
                return carry
            lax.fori_loop(0, nfull, step, 0)
        if tail:
            base = base_w + nfull * _K
            pltpu.sync_copy(idx_hbm.at[pl.ds(base, tail)], idx_t)
            pltpu.sync_copy(tab_hbm.at[idx_t], rows_t)
            pltpu.sync_copy(rows_t, out_hbm.at[pl.ds(base, tail)])

    scratch_types = [
        pltpu.VMEM((_K,), jnp.int32),
        pltpu.VMEM((_K, ch), jnp.float32),
    ]
    if tail:
        scratch_types += [
            pltpu.VMEM((tail,), jnp.int32),
            pltpu.VMEM((tail, ch), jnp.float32),
        ]

    return pl.kernel(
        body,
        out_type=jax.ShapeDtypeStruct((np_out, ch), jnp.float32),
        mesh=_mesh(),
        scratch_types=scratch_types,
        compiler_params=pltpu.CompilerParams(use_tc_tiling_on_sc=False),
        name=f"sc_row_gather_n{np_out}_c{ch}",
    )


@functools.lru_cache(maxsize=None)
def _tc_fused(rows, co, mats, nplain, aggmode, aggw, bias, bn, res, relu,
              skip_ci):
    if rows <= 1280:
        bn_rows = rows
    elif rows == 2560:
        bn_rows = 1280
    else:
        bn_rows = 1024
    grid = (rows + bn_rows - 1) // bn_rows

    n_x = len(mats)

    def body(*refs):
        it = iter(refs)
        x_refs = [next(it) for _ in range(n_x)]
        plain_refs = [next(it) for _ in range(nplain)]
        agg_ref = deg_ref = None
        if aggmode:
            agg_ref = next(it)
            deg_ref = next(it)
        w_refs = [next(it) for _ in range(n_x)]
        wn_ref = next(it) if aggmode == "mm" else None
        b_ref = next(it) if bias else None
        g_ref = next(it) if bn else None
        be_ref = next(it) if bn else None
        res_ref = next(it) if res else None
        wsk_ref = next(it) if skip_ci is not None else None
        out_ref = next(it)
        s_ref = next(it) if skip_ci is not None else None

        acc = jnp.zeros((bn_rows, co), jnp.float32)
        for xr, wr in zip(x_refs, w_refs):
            acc = acc + jnp.dot(xr[...], wr[...],
                                preferred_element_type=jnp.float32)
        for pr in plain_refs:
            acc = acc + pr[...]
        if aggmode:
            a = agg_ref[0] + agg_ref[1]
            dg = deg_ref[0] + deg_ref[1]
            inv = 1.0 / jnp.maximum(dg[:, :1], 1.0)
            m = a * inv
            if aggmode == "mm":
                acc = acc + jnp.dot(m, wn_ref[...],
                                    preferred_element_type=jnp.float32)
            else:
                acc = acc + m
        if bias:
            acc = acc + b_ref[...]
        if bn:
            acc = acc * g_ref[...] + be_ref[...]
        if res:
            acc = acc + res_ref[...]
        if relu:
            acc = jnp.maximum(acc, 0.0)
        out_ref[...] = acc
        if skip_ci is not None:
            s_ref[...] = jnp.dot(x_refs[0][...], wsk_ref[...],
                                 preferred_element_type=jnp.float32)

    in_specs = []
    for ci in mats:
        in_specs.append(pl.BlockSpec((bn_rows, ci), lambda i: (i, 0)))
    for _ in range(nplain):
        in_specs.append(pl.BlockSpec((bn_rows, co), lambda i: (i, 0)))
    if aggmode:
        in_specs.append(pl.BlockSpec((2, bn_rows, aggw), lambda i: (0, i, 0)))
        in_specs.append(pl.BlockSpec((2, bn_rows, 8), lambda i: (0, i, 0)))
    for ci in mats:
        in_specs.append(pl.BlockSpec((ci, co), lambda i: (0, 0)))
    if aggmode == "mm":
        in_specs.append(pl.BlockSpec((aggw, co), lambda i: (0, 0)))
    if bias:
        in_specs.append(pl.BlockSpec((1, co), lambda i: (0, 0)))
    if bn:
        in_specs.append(pl.BlockSpec((1, co), lambda i: (0, 0)))
        in_specs.append(pl.BlockSpec((1, co), lambda i: (0, 0)))
    if res:
        in_specs.append(pl.BlockSpec((bn_rows, co), lambda i: (i, 0)))
    if skip_ci is not None:
        in_specs.append(pl.BlockSpec((skip_ci, co), lambda i: (0, 0)))

    out_shape = [jax.ShapeDtypeStruct((rows, co), jnp.float32)]
    out_specs = [pl.BlockSpec((bn_rows, co), lambda i: (i, 0))]
    if skip_ci is not None:
        out_shape.append(jax.ShapeDtypeStruct((rows, co), jnp.float32))
        out_specs.append(pl.BlockSpec((bn_rows, co), lambda i: (i, 0)))

    return pl.pallas_call(
        body,
        grid=(grid,),
        in_specs=in_specs,
        out_specs=out_specs,
        out_shape=out_shape,
        name=f"tc_fused_r{rows}_co{co}_m{mats}_p{nplain}_{aggmode}{aggw}",
    )


def _tc_op(xw, plains=(), agg=None, wn=None, b=None, gamma=None, beta=None,
           res=None, relu=False, skip_w=None, deg=None):
    mats = tuple(int(w.shape[0]) for _, w in xw)
    if xw:
        co = int(xw[0][1].shape[1])
        rows = int(xw[0][0].shape[0])
    elif plains:
        co = int(plains[0].shape[1])
        rows = int(plains[0].shape[0])
    else:
        co = int(wn.shape[1]) if wn is not None else int(agg.shape[2])
        rows = int(agg.shape[1])
    aggmode = ""
    aggw = 0
    if agg is not None:
        aggmode = "mm" if wn is not None else "raw"
        aggw = int(agg.shape[2])
    fn = _tc_fused(rows, co, mats, len(plains), aggmode, aggw,
                   b is not None, gamma is not None, res is not None,
                   relu, None if skip_w is None else int(skip_w.shape[0]))
    args = [x for x, _ in xw] + list(plains)
    if agg is not None:
        args += [agg, deg]
    args += [w for _, w in xw]
    if aggmode == "mm":
        args.append(wn)
    if b is not None:
        args.append(b.reshape(1, -1))
    if gamma is not None:
        args += [gamma.reshape(1, -1), beta.reshape(1, -1)]
    if res is not None:
        args.append(res)
    if skip_w is not None:
        args.append(skip_w)
    out = fn(*args)
    return out if skip_w is not None else out[0]


@functools.lru_cache(maxsize=None)
def _tc_header(rows_in, n_out, c1, c2):
    bn_rows = 1000
    grid = n_out // bn_rows

    def body(x_ref, w1, b1, g1, be1, w2, b2, out_ref):
        h = jnp.dot(x_ref[...], w1[...], preferred_element_type=jnp.float32)
        h = (h + b1[...]) * g1[...] + be1[...]
        h = jnp.maximum(h, 0.0)
        out_ref[...] = jnp.dot(h, w2[...],
                               preferred_element_type=jnp.float32) + b2[...]

    cin = c1
    in_specs = [
        pl.BlockSpec((bn_rows, cin), lambda i: (i, 0)),
        pl.BlockSpec((cin, c1), lambda i: (0, 0)),
        pl.BlockSpec((1, c1), lambda i: (0, 0)),
        pl.BlockSpec((1, c1), lambda i: (0, 0)),
        pl.BlockSpec((1, c1), lambda i: (0, 0)),
        pl.BlockSpec((c1, c2), lambda i: (0, 0)),
        pl.BlockSpec((1, c2), lambda i: (0, 0)),
    ]
    return pl.pallas_call(
        body,
        grid=(grid,),
        in_specs=in_specs,
        out_specs=pl.BlockSpec((bn_rows, c2), lambda i: (i, 0)),
        out_shape=jax.ShapeDtypeStruct((n_out, c2), jnp.float32),
        name="tc_header",
    )


def _edge_pad(e, level):
    ep = _round_up(_ES[level], _NWORK * _K)
    n = e.shape[1]
    src = jnp.pad(e[0], (0, ep - n)).reshape(ep // _K, _K)
    dst = jnp.pad(e[1], (0, ep - n),
                  constant_values=_NS[level]).reshape(ep // _K, _K)
    return src, dst, ep


def kernel(data, params, edge_index_0, edge_index_1, edge_index_2,
           edge_index_3, edge_index_4, cluster_0, cluster_1, cluster_2,
           cluster_3, depth):
    del depth
    edges = [edge_index_0, edge_index_1, edge_index_2, edge_index_3,
             edge_index_4]
    clusters = [cluster_0, cluster_1, cluster_2, cluster_3]

    epad = [_edge_pad(edges[i], i) for i in range(5)]
    clp = [jnp.pad(clusters[i], (0, _NPAD[i] - _NS[i]),
                   constant_values=_NS[i + 1]) for i in range(4)]

    def agg(level, x):
        src, dst, ep = epad[level]
        ch = int(x.shape[1])
        f = _sc_edge_agg(ep, _NPAD[level], ch)
        return f(x, src, dst, jnp.zeros((_NPAD[level], ch), jnp.float32))

    degp = {}

    def agg_d(level, x):
        if level in degp:
            return agg(level, x)
        ch = int(x.shape[1])
        xa = jnp.concatenate(
            [x, jnp.ones((x.shape[0], 16), jnp.float32)], axis=1)
        full = agg(level, xa)
        degp[level] = full[:, :, ch:ch + 8]
        return full[:, :, :ch]

    def pool_sum(level, x):
        ch = int(x.shape[1])
        f = _sc_seg_sum(_NPAD[level], _NPAD[level + 1], ch, False)
        return f(x, clp[level], jnp.zeros((_NPAD[level + 1], ch),
                                          jnp.float32))

    def unpool(level, table):
        ch = int(table.shape[1])
        f = _sc_row_gather(_NPAD[level], ch)
        return f(table, clp[level])

    def gconv_in(level, x, p, res=None, relu=True, skip_w=None):
        ap = agg_d(level, x)
        return _tc_op([(x, p["Ws"])], agg=ap, wn=p["Wn"], b=p["b"],
                      gamma=p["gamma"], beta=p["beta"], res=res, relu=relu,
                      skip_w=skip_w, deg=degp[level])

    def gconv_proj(level, x_parts, p, skip_w=None, relu=True):
        co = int(p["Ws"].shape[1])
        stack = [p["Wn"], p["Ws"]] + ([skip_w] if skip_w is not None else [])
        wstack = jnp.concatenate(stack, axis=1)
        row_off = 0
        xw = []
        for x in x_parts:
            ci = int(x.shape[1])
            xw.append((x, wstack[row_off:row_off + ci]))
            row_off += ci
        proj = _tc_op(xw)
        xn = proj[:, :co]
        xs = proj[:, co:2 * co]
        s = proj[:, 2 * co:] if skip_w is not None else None
        ap = agg_d(level, xn)
        h = _tc_op([], plains=(xs,), agg=ap, b=p["b"], gamma=p["gamma"],
                   beta=p["beta"], relu=relu, deg=degp[level])
        return h, s

    def resblock(level, x_parts, p, ci, co):
        skip_w = p.get("skip")
        if len(x_parts) == 1 and ci <= co:
            x = x_parts[0]
            if skip_w is not None:
                h, s = gconv_in(level, x, p["c1"], skip_w=skip_w)
            else:
                h = gconv_in(level, x, p["c1"])
                s = x
        else:
            h, s = gconv_proj(level, x_parts, p["c1"], skip_w=skip_w)
            if s is None:
                s = x_parts[0]
        return gconv_in(level, h, p["c2"], res=s, relu=True)

    def stage(level, x_parts, blks, ci, co):
        x = resblock(level, x_parts, blks[0], ci, co)
        for p in blks[1:]:
            x = resblock(level, [x], p, co, co)
        return x

    enc_ch = [32, 32, 64, 64, 128]
    dec_ch = [128, 64, 64, 32, 32]

    data_p = jnp.pad(data, ((0, _NPAD[0] - _NS[0]), (0, 0)))
    conv1, _ = gconv_proj(0, [data_p], params["conv1"])
    convd = {0: conv1}

    for i in range(4):
        ch_i = int(convd[i].shape[1])
        vals = jnp.concatenate(
            [convd[i], jnp.ones((_NPAD[i], 16), jnp.float32)], axis=1)
        full = pool_sum(i, vals)
        psum = full[:, :, :ch_i]
        cnt = full[:, :, ch_i:ch_i + 8]
        pooled = _tc_op([], agg=psum, deg=cnt)
        convd[i + 1] = stage(i + 1, [pooled], params["enc"][i],
                             enc_ch[i], enc_ch[i + 1])

    deconv = convd[4]
    for i in range(4):
        fine = 3 - i
        g = unpool(fine, deconv)
        deconv = stage(fine, [convd[fine], g], params["dec"][i],
                       enc_ch[fine + 1] + dec_ch[i], dec_ch[i + 1])

    h1, h2 = params["header1"], params["header2"]
    hdr = _tc_header(_NPAD[0], _NS[0], int(h1["W"].shape[0]),
                     int(h2["W"].shape[1]))
    return hdr(deconv, h1["W"], h1["b"].reshape(1, -1),
               h1["gamma"].reshape(1, -1), h1["beta"].reshape(1, -1),
               h2["W"], h2["b"].reshape(1, -1))


# speedup vs baseline: 1.0006x; 1.0006x over previous
import functools

import jax
import jax.numpy as jnp
from jax import lax
from jax.experimental import pallas as pl
from jax.experimental.pallas import tpu as pltpu
from jax.experimental.pallas import tpu_sc as plsc

_NS = [10000, 5000, 2500, 1250, 625]
_ES = [320000, 160000, 80000, 40000, 20000]
_NPAD = [10240, 5120, 2560, 1280, 640]

_NCORE, _NSUB = 2, 16
_NWORK = _NCORE * _NSUB
_K = 128


def _round_up(x, m):
    return (x + m - 1) // m * m


def _mesh():
    return plsc.VectorSubcoreMesh(
        core_axis_name="c", subcore_axis_name="s",
        num_cores=_NCORE, num_subcores=_NSUB)


@functools.lru_cache(maxsize=None)
def _sc_edge_agg(ep, np_dst, ch):
    nrows = ep // _K
    rw = nrows // _NWORK
    rows_sub = np_dst // _NSUB
    _NB = 8 if ch <= 80 else 4
    _AH = _NB // 2

    def body(*refs):
        x_hbm, src_hbm, dst_hbm, zeros_hbm, out_hbm, sidx, didx = refs[:7]
        rows = list(refs[7:7 + _NB])
        gs = list(refs[7 + _NB:7 + 2 * _NB])
        ss = list(refs[7 + 2 * _NB:7 + 3 * _NB])
        acc = refs[7 + 3 * _NB]
        c = lax.axis_index("c")
        s = lax.axis_index("s")
        w = c * _NSUB + s
        pltpu.sync_copy(zeros_hbm.at[pl.ds(s * rows_sub, rows_sub)],
                        acc.at[pl.ds(s * rows_sub, rows_sub)])
        pltpu.sync_copy(src_hbm.at[pl.ds(w * rw, rw)], sidx)
        pltpu.sync_copy(dst_hbm.at[pl.ds(w * rw, rw)], didx)
        plsc.subcore_barrier()

        def g_start(j, b):
            pltpu.async_copy(x_hbm.at[sidx.at[j]], rows[b], gs[b])

        def g_wait(j, b):
            pltpu.make_async_copy(x_hbm.at[sidx.at[j]], rows[b],
                                  gs[b]).wait()

        def s_start(j, b):
            pltpu.async_copy(rows[b], acc.at[didx.at[j]], ss[b], add=True)

        def s_wait(j, b):
            pltpu.make_async_copy(rows[b], acc.at[didx.at[j]],
                                  ss[b]).wait()

        def step(j, b, static_j=None):
            g_wait(j, b)
            s_start(j, b)
            jm = j - _AH
            jp = j + _AH
            if static_j is None:
                @pl.when(jm >= 0)
                def _():
                    s_wait(jm, (b - _AH) % _NB)

                @pl.when(jp < rw)
                def _():
                    g_start(jp, (b + _AH) % _NB)
            else:
                if static_j - _AH >= 0:
                    s_wait(jm, (b - _AH) % _NB)
                if static_j + _AH < rw:
                    g_start(jp, (b + _AH) % _NB)

        for b0 in range(min(_AH, rw)):
            g_start(b0, b0)
        nq = rw // _NB

        def quad(q, carry):
            j0 = q * _NB
            for b in range(_NB):
                step(j0 + b, b)
            return carry
        lax.fori_loop(0, nq, quad, 0)
        for j in range(nq * _NB, rw):
            step(j, j % _NB, static_j=j)
        for j in range(max(0, rw - _AH), rw):
            s_wait(j, j % _NB)

        plsc.subcore_barrier()
        pltpu.sync_copy(acc.at[pl.ds(s * rows_sub, rows_sub)],
                        out_hbm.at[c, pl.ds(s * rows_sub, rows_sub)])

    scratch_types = (
        [pltpu.VMEM((rw, _K), jnp.int32)] * 2
        + [pltpu.VMEM((_K, ch), jnp.float32)] * _NB
        + [pltpu.SemaphoreType.DMA] * (2 * _NB)
        + [pltpu.VMEM_SHARED((np_dst, ch), jnp.float32)]
    )

    return pl.kernel(
        body,
        out_type=jax.ShapeDtypeStruct((2, np_dst, ch), jnp.float32),
        mesh=_mesh(),
        scratch_types=scratch_types,
        compiler_params=pltpu.CompilerParams(use_tc_tiling_on_sc=False),
        name=f"sc_edge_agg_e{ep}_n{np_dst}_c{ch}",
    )


@functools.lru_cache(maxsize=None)
def _sc_seg_sum(ep, np_dst, ch, gather):
    ew = ep // _NWORK
    nfull, tail = ew // _K, ew % _K
    rows_sub = np_dst // _NSUB

    def body(*refs):
        if gather:
            x_hbm, src_hbm, dst_hbm, zeros_hbm, out_hbm = refs[:5]
            scratch = refs[5:]
        else:
            x_hbm, dst_hbm, zeros_hbm, out_hbm = refs[:4]
            src_hbm = None
            scratch = refs[4:]
        sidx_v, didx_v, rows_v = scratch[:3]
        scratch = scratch[3:]
        if tail:
            sidx_t, didx_t, rows_t = scratch[:3]
            scratch = scratch[3:]
        acc = scratch[0]

        c = lax.axis_index("c")
        s = lax.axis_index("s")
        base_w = (c * _NSUB + s) * ew

        pltpu.sync_copy(zeros_hbm.at[pl.ds(s * rows_sub, rows_sub)],
                        acc.at[pl.ds(s * rows_sub, rows_sub)])
        plsc.subcore_barrier()

        if nfull:
            def step(j, carry):
                base = base_w + j * _K
                pltpu.sync_copy(dst_hbm.at[pl.ds(base, _K)], didx_v)
                if gather:
                    pltpu.sync_copy(src_hbm.at[pl.ds(base, _K)], sidx_v)
                    pltpu.sync_copy(x_hbm.at[sidx_v], rows_v)
                else:
                    pltpu.sync_copy(x_hbm.at[pl.ds(base, _K)], rows_v)
                pltpu.sync_copy(rows_v, acc.at[didx_v], add=True)
                return carry
            lax.fori_loop(0, nfull, step, 0)
        if tail:
            base = base_w + nfull * _K
            pltpu.sync_copy(dst_hbm.at[pl.ds(base, tail)], didx_t)
            if gather:
                pltpu.sync_copy(src_hbm.at[pl.ds(base, tail)], sidx_t)
                pltpu.sync_copy(x_hbm.at[sidx_t], rows_t)
            else:
                pltpu.sync_copy(x_hbm.at[pl.ds(base, tail)], rows_t)
            pltpu.sync_copy(rows_t, acc.at[didx_t], add=True)

        plsc.subcore_barrier()
        pltpu.sync_copy(acc.at[pl.ds(s * rows_sub, rows_sub)],
                        out_hbm.at[c, pl.ds(s * rows_sub, rows_sub)])

    scratch_types = [
        pltpu.VMEM((_K,), jnp.int32),
        pltpu.VMEM((_K,), jnp.int32),
        pltpu.VMEM((_K, ch), jnp.float32),
    ]
    if tail:
        scratch_types += [
            pltpu.VMEM((tail,), jnp.int32),
            pltpu.VMEM((tail,), jnp.int32),
            pltpu.VMEM((tail, ch), jnp.float32),
        ]
    scratch_types.append(pltpu.VMEM_SHARED((np_dst, ch), jnp.float32))

    return pl.kernel(
        body,
        out_type=jax.ShapeDtypeStruct((2, np_dst, ch), jnp.float32),
        mesh=_mesh(),
        scratch_types=scratch_types,
        compiler_params=pltpu.CompilerParams(use_tc_tiling_on_sc=False),
        name=f"sc_seg_sum_e{ep}_n{np_dst}_c{ch}_{int(gather)}",
    )


@functools.lru_cache(maxsize=None)
def _sc_row_gather(np_out, ch):
    ew = np_out // _NWORK
    nfull, tail = ew // _K, ew % _K

    def body(*refs):
        tab_hbm, idx_hbm, out_hbm = refs[:3]
        scratch = refs[3:]
        idx_v, rows_v = scratch[:2]
        if tail:
            idx_t, rows_t = scratch[2:4]
        c = lax.axis_index("c")
        s = lax.axis_index("s")
        base_w = (c * _NSUB + s) * ew
        if nfull:
            def step(j, carry):
                base = base_w + j * _K
                pltpu.sync_copy(idx_hbm.at[pl.ds(base, _K)], idx_v)
                pltpu.sync_copy(tab_hbm.at[idx_v], rows_v)
                pltpu.sync_copy(rows_v, out_hbm.at[pl.ds(base, _K)])
                return carry
            lax.fori_loop(0, nfull, step, 0)
        if tail:
            base = base_w + nfull * _K
            pltpu.sync_copy(idx_hbm.at[pl.ds(base, tail)], idx_t)
            pltpu.sync_copy(tab_hbm.at[idx_t], rows_t)
            pltpu.sync_copy(rows_t, out_hbm.at[pl.ds(base, tail)])

    scratch_types = [
        pltpu.VMEM((_K,), jnp.int32),
        pltpu.VMEM((_K, ch), jnp.float32),
    ]
    if tail:
        scratch_types += [
            pltpu.VMEM((tail,), jnp.int32),
            pltpu.VMEM((tail, ch), jnp.float32),
        ]

    return pl.kernel(
        body,
        out_type=jax.ShapeDtypeStruct((np_out, ch), jnp.float32),
        mesh=_mesh(),
        scratch_types=scratch_types,
        compiler_params=pltpu.CompilerParams(use_tc_tiling_on_sc=False),
        name=f"sc_row_gather_n{np_out}_c{ch}",
    )


@functools.lru_cache(maxsize=None)
def _tc_fused(rows, co, mats, nplain, aggmode, aggw, bias, bn, res, relu,
              skip_ci):
    if rows <= 1280:
        bn_rows = rows
    elif rows == 2560:
        bn_rows = 1280
    else:
        bn_rows = 1024
    grid = (rows + bn_rows - 1) // bn_rows

    n_x = len(mats)

    def body(*refs):
        it = iter(refs)
        x_refs = [next(it) for _ in range(n_x)]
        plain_refs = [next(it) for _ in range(nplain)]
        agg_ref = deg_ref = None
        if aggmode:
            agg_ref = next(it)
            deg_ref = next(it)
        w_refs = [next(it) for _ in range(n_x)]
        wn_ref = next(it) if aggmode == "mm" else None
        b_ref = next(it) if bias else None
        g_ref = next(it) if bn else None
        be_ref = next(it) if bn else None
        res_ref = next(it) if res else None
        wsk_ref = next(it) if skip_ci is not None else None
        out_ref = next(it)
        s_ref = next(it) if skip_ci is not None else None

        acc = jnp.zeros((bn_rows, co), jnp.float32)
        for xr, wr in zip(x_refs, w_refs):
            acc = acc + jnp.dot(xr[...], wr[...],
                                preferred_element_type=jnp.float32)
        for pr in plain_refs:
            acc = acc + pr[...]
        if aggmode:
            a = agg_ref[0] + agg_ref[1]
            dg = deg_ref[0] + deg_ref[1]
            inv = 1.0 / jnp.maximum(dg[:, :1], 1.0)
            m = a * inv
            if aggmode == "mm":
                acc = acc + jnp.dot(m, wn_ref[...],
                                    preferred_element_type=jnp.float32)
            else:
                acc = acc + m
        if bias:
            acc = acc + b_ref[...]
        if bn:
            acc = acc * g_ref[...] + be_ref[...]
        if res:
            acc = acc + res_ref[...]
        if relu:
            acc = jnp.maximum(acc, 0.0)
        out_ref[...] = acc
        if skip_ci is not None:
            s_ref[...] = jnp.dot(x_refs[0][...], wsk_ref[...],
                                 preferred_element_type=jnp.float32)

    in_specs = []
    for ci in mats:
        in_specs.append(pl.BlockSpec((bn_rows, ci), lambda i: (i, 0)))
    for _ in range(nplain):
        in_specs.append(pl.BlockSpec((bn_rows, co), lambda i: (i, 0)))
    if aggmode:
        in_specs.append(pl.BlockSpec((2, bn_rows, aggw), lambda i: (0, i, 0)))
        in_specs.append(pl.BlockSpec((2, bn_rows, 8), lambda i: (0, i, 0)))
    for ci in mats:
        in_specs.append(pl.BlockSpec((ci, co), lambda i: (0, 0)))
    if aggmode == "mm":
        in_specs.append(pl.BlockSpec((aggw, co), lambda i: (0, 0)))
    if bias:
        in_specs.append(pl.BlockSpec((co,), lambda i: (0,)))
    if bn:
        in_specs.append(pl.BlockSpec((co,), lambda i: (0,)))
        in_specs.append(pl.BlockSpec((co,), lambda i: (0,)))
    if res:
        in_specs.append(pl.BlockSpec((bn_rows, co), lambda i: (i, 0)))
    if skip_ci is not None:
        in_specs.append(pl.BlockSpec((skip_ci, co), lambda i: (0, 0)))

    out_shape = [jax.ShapeDtypeStruct((rows, co), jnp.float32)]
    out_specs = [pl.BlockSpec((bn_rows, co), lambda i: (i, 0))]
    if skip_ci is not None:
        out_shape.append(jax.ShapeDtypeStruct((rows, co), jnp.float32))
        out_specs.append(pl.BlockSpec((bn_rows, co), lambda i: (i, 0)))

    return pl.pallas_call(
        body,
        grid=(grid,),
        in_specs=in_specs,
        out_specs=out_specs,
        out_shape=out_shape,
        name=f"tc_fused_r{rows}_co{co}_m{mats}_p{nplain}_{aggmode}{aggw}",
    )


def _tc_op(xw, plains=(), agg=None, wn=None, b=None, gamma=None, beta=None,
           res=None, relu=False, skip_w=None, deg=None):
    mats = tuple(int(w.shape[0]) for _, w in xw)
    if xw:
        co = int(xw[0][1].shape[1])
        rows = int(xw[0][0].shape[0])
    elif plains:
        co = int(plains[0].shape[1])
        rows = int(plains[0].shape[0])
    else:
        co = int(wn.shape[1]) if wn is not None else int(agg.shape[2])
        rows = int(agg.shape[1])
    aggmode = ""
    aggw = 0
    if agg is not None:
        aggmode = "mm" if wn is not None else "raw"
        aggw = int(agg.shape[2])
    fn = _tc_fused(rows, co, mats, len(plains), aggmode, aggw,
                   b is not None, gamma is not None, res is not None,
                   relu, None if skip_w is None else int(skip_w.shape[0]))
    args = [x for x, _ in xw] + list(plains)
    if agg is not None:
        args += [agg, deg]
    args += [w for _, w in xw]
    if aggmode == "mm":
        args.append(wn)
    if b is not None:
        args.append(b)
    if gamma is not None:
        args += [gamma, beta]
    if res is not None:
        args.append(res)
    if skip_w is not None:
        args.append(skip_w)
    out = fn(*args)
    return out if skip_w is not None else out[0]


@functools.lru_cache(maxsize=None)
def _tc_header(rows_in, n_out, c1, c2):
    bn_rows = 1000
    grid = n_out // bn_rows

    def body(x_ref, w1, b1, g1, be1, w2, b2, out_ref):
        h = jnp.dot(x_ref[...], w1[...], preferred_element_type=jnp.float32)
        h = (h + b1[...]) * g1[...] + be1[...]
        h = jnp.maximum(h, 0.0)
        out_ref[...] = jnp.dot(h, w2[...],
                               preferred_element_type=jnp.float32) + b2[...]

    cin = c1
    in_specs = [
        pl.BlockSpec((bn_rows, cin), lambda i: (i, 0)),
        pl.BlockSpec((cin, c1), lambda i: (0, 0)),
        pl.BlockSpec((c1,), lambda i: (0,)),
        pl.BlockSpec((c1,), lambda i: (0,)),
        pl.BlockSpec((c1,), lambda i: (0,)),
        pl.BlockSpec((c1, c2), lambda i: (0, 0)),
        pl.BlockSpec((c2,), lambda i: (0,)),
    ]
    return pl.pallas_call(
        body,
        grid=(grid,),
        in_specs=in_specs,
        out_specs=pl.BlockSpec((bn_rows, c2), lambda i: (i, 0)),
        out_shape=jax.ShapeDtypeStruct((n_out, c2), jnp.float32),
        name="tc_header",
    )


def _edge_pad(e, level):
    ep = _round_up(_ES[level], _NWORK * _K)
    n = e.shape[1]
    src = jnp.pad(e[0], (0, ep - n)).reshape(ep // _K, _K)
    dst = jnp.pad(e[1], (0, ep - n),
                  constant_values=_NS[level]).reshape(ep // _K, _K)
    return src, dst, ep


def kernel(data, params, edge_index_0, edge_index_1, edge_index_2,
           edge_index_3, edge_index_4, cluster_0, cluster_1, cluster_2,
           cluster_3, depth):
    del depth
    edges = [edge_index_0, edge_index_1, edge_index_2, edge_index_3,
             edge_index_4]
    clusters = [cluster_0, cluster_1, cluster_2, cluster_3]

    epad = [_edge_pad(edges[i], i) for i in range(5)]
    clp = [jnp.pad(clusters[i], (0, _NPAD[i] - _NS[i]),
                   constant_values=_NS[i + 1]) for i in range(4)]

    def agg(level, x):
        src, dst, ep = epad[level]
        ch = int(x.shape[1])
        f = _sc_edge_agg(ep, _NPAD[level], ch)
        return f(x, src, dst, jnp.zeros((_NPAD[level], ch), jnp.float32))

    degp = {}

    def agg_d(level, x):
        if level in degp:
            return agg(level, x)
        ch = int(x.shape[1])
        xa = jnp.concatenate(
            [x, jnp.ones((x.shape[0], 16), jnp.float32)], axis=1)
        full = agg(level, xa)
        degp[level] = full[:, :, ch:ch + 8]
        return full[:, :, :ch]

    def pool_sum(level, x):
        ch = int(x.shape[1])
        f = _sc_seg_sum(_NPAD[level], _NPAD[level + 1], ch, False)
        return f(x, clp[level], jnp.zeros((_NPAD[level + 1], ch),
                                          jnp.float32))

    def unpool(level, table):
        ch = int(table.shape[1])
        f = _sc_row_gather(_NPAD[level], ch)
        return f(table, clp[level])

    def gconv_in(level, x, p, res=None, relu=True, skip_w=None):
        ap = agg_d(level, x)
        return _tc_op([(x, p["Ws"])], agg=ap, wn=p["Wn"], b=p["b"],
                      gamma=p["gamma"], beta=p["beta"], res=res, relu=relu,
                      skip_w=skip_w, deg=degp[level])

    def gconv_proj(level, x_parts, p, skip_w=None, relu=True):
        co = int(p["Ws"].shape[1])
        stack = [p["Wn"], p["Ws"]] + ([skip_w] if skip_w is not None else [])
        wstack = jnp.concatenate(stack, axis=1)
        row_off = 0
        xw = []
        for x in x_parts:
            ci = int(x.shape[1])
            xw.append((x, wstack[row_off:row_off + ci]))
            row_off += ci
        proj = _tc_op(xw)
        xn = proj[:, :co]
        xs = proj[:, co:2 * co]
        s = proj[:, 2 * co:] if skip_w is not None else None
        ap = agg_d(level, xn)
        h = _tc_op([], plains=(xs,), agg=ap, b=p["b"], gamma=p["gamma"],
                   beta=p["beta"], relu=relu, deg=degp[level])
        return h, s

    def resblock(level, x_parts, p, ci, co):
        skip_w = p.get("skip")
        if len(x_parts) == 1 and ci <= co:
            x = x_parts[0]
            if skip_w is not None:
                h, s = gconv_in(level, x, p["c1"], skip_w=skip_w)
            else:
                h = gconv_in(level, x, p["c1"])
                s = x
        else:
            h, s = gconv_proj(level, x_parts, p["c1"], skip_w=skip_w)
            if s is None:
                s = x_parts[0]
        return gconv_in(level, h, p["c2"], res=s, relu=True)

    def stage(level, x_parts, blks, ci, co):
        x = resblock(level, x_parts, blks[0], ci, co)
        for p in blks[1:]:
            x = resblock(level, [x], p, co, co)
        return x

    enc_ch = [32, 32, 64, 64, 128]
    dec_ch = [128, 64, 64, 32, 32]

    data_p = jnp.pad(data, ((0, _NPAD[0] - _NS[0]), (0, 0)))
    conv1, _ = gconv_proj(0, [data_p], params["conv1"])
    convd = {0: conv1}

    for i in range(4):
        ch_i = int(convd[i].shape[1])
        vals = jnp.concatenate(
            [convd[i], jnp.ones((_NPAD[i], 16), jnp.float32)], axis=1)
        full = pool_sum(i, vals)
        psum = full[:, :, :ch_i]
        cnt = full[:, :, ch_i:ch_i + 8]
        pooled = _tc_op([], agg=psum, deg=cnt)
        convd[i + 1] = stage(i + 1, [pooled], params["enc"][i],
                             enc_ch[i], enc_ch[i + 1])

    deconv = convd[4]
    for i in range(4):
        fine = 3 - i
        g = unpool(fine, deconv)
        deconv = stage(fine, [convd[fine], g], params["dec"][i],
                       enc_ch[fine + 1] + dec_ch[i], dec_ch[i + 1])

    h1, h2 = params["header1"], params["header2"]
    hdr = _tc_header(_NPAD[0], _NS[0], int(h1["W"].shape[0]),
                     int(h2["W"].shape[1]))
    return hdr(deconv, h1["W"], h1["b"], h1["gamma"], h1["beta"],
               h2["W"], h2["b"])
